# Initial kernel scaffold; baseline (speedup 1.0000x reference)
import jax
import jax.numpy as jnp
from jax.experimental import pallas as pl


def kernel(x, edge_index, edge_length, edge_attr, W1, mlp_w1, mlp_b1, mlp_w2, mlp_b2, W2, b2, Wlin, blin):
    raise NotImplementedError("write your pallas kernel here")


# speedup vs baseline: 1.3223x; 1.3223x over previous
import functools

import jax
import jax.numpy as jnp
from jax import lax
from jax.experimental import pallas as pl
from jax.experimental.pallas import tpu as pltpu
from jax.experimental.pallas import tpu_sc as plsc

N = 10000
E = 320000
H = 128
G = 50
F = 128
CUTOFF = 10.0
SHIFT = 0.6931471805599453

NC = 2
NS = 16
NW = NC * NS
K = 80
EPW = E // NW
CH = EPW // K

BE = 2000
BN = 2000


def _ssp(v):
    return jnp.maximum(v, 0.0) + jnp.log1p(jnp.exp(-jnp.abs(v))) - SHIFT


def _wfilt_body(ea_ref, el_ref, w1t_ref, b1_ref, w2t_ref, b2_ref, out_ref):
    hid = _ssp(jnp.dot(ea_ref[...], w1t_ref[...],
                       preferred_element_type=jnp.float32) + b1_ref[...])
    wf = jnp.dot(hid, w2t_ref[...],
                 preferred_element_type=jnp.float32) + b2_ref[...]
    el = el_ref[...]
    c = 0.5 * (jnp.cos(el * (jnp.pi / CUTOFF)) + 1.0)
    c = c * (el <= CUTOFF).astype(jnp.float32) * (el >= 0.0).astype(jnp.float32)
    out_ref[...] = wf * c


def _wfilt(edge_attr, edge_length, mlp_w1, mlp_b1, mlp_w2, mlp_b2):
    grid = (E // BE,)
    return pl.pallas_call(
        _wfilt_body,
        grid=grid,
        in_specs=[
            pl.BlockSpec((BE, G), lambda i: (i, 0)),
            pl.BlockSpec((BE, 1), lambda i: (i, 0)),
            pl.BlockSpec((G, F), lambda i: (0, 0)),
            pl.BlockSpec((1, F), lambda i: (0, 0)),
            pl.BlockSpec((F, F), lambda i: (0, 0)),
            pl.BlockSpec((1, F), lambda i: (0, 0)),
        ],
        out_specs=pl.BlockSpec((BE, F), lambda i: (i, 0)),
        out_shape=jax.ShapeDtypeStruct((E, F), jnp.float32),
    )(edge_attr, edge_length.reshape(E, 1), mlp_w1.T, mlp_b1.reshape(1, F),
      mlp_w2.T, mlp_b2.reshape(1, F))


def _h_body(x_ref, w_ref, out_ref):
    out_ref[...] = jnp.dot(x_ref[...], w_ref[...],
                           preferred_element_type=jnp.float32)


def _h(x, W1):
    return pl.pallas_call(
        _h_body,
        grid=(N // BN,),
        in_specs=[
            pl.BlockSpec((BN, H), lambda i: (i, 0)),
            pl.BlockSpec((H, F), lambda i: (0, 0)),
        ],
        out_specs=pl.BlockSpec((BN, F), lambda i: (i, 0)),
        out_shape=jax.ShapeDtypeStruct((N, F), jnp.float32),
    )(x, W1.T)


def _sc_body(h_hbm, wf_hbm, src_hbm, dst_hbm, zero_hbm,
             mij_hbm, part_hbm,
             srcv, dstv, hv, wv, acc, sem):
    cid = lax.axis_index("c")
    sid = lax.axis_index("s")
    wid = sid * NC + cid

    rpt = 624
    pltpu.sync_copy(zero_hbm.at[pl.ds(sid * rpt, rpt)],
                    acc.at[pl.ds(sid * rpt, rpt)])
    @pl.when(sid == NS - 1)
    def _():
        pltpu.sync_copy(zero_hbm.at[pl.ds(NS * rpt, N - NS * rpt)],
                        acc.at[pl.ds(NS * rpt, N - NS * rpt)])
    plsc.subcore_barrier()

    base_w = wid * EPW

    def chunk(j, carry):
        base = base_w + j * K
        pltpu.sync_copy(src_hbm.at[pl.ds(base, K)], srcv)
        pltpu.sync_copy(dst_hbm.at[pl.ds(base, K)], dstv)
        pltpu.async_copy(h_hbm.at[srcv], hv, sem).wait()
        pltpu.sync_copy(wf_hbm.at[pl.ds(base, K)], wv)

        def row(r, c2):
            for g in range(F // 16):
                sl = pl.ds(g * 16, 16)
                hv[r, sl] = hv[r, sl] * wv[r, sl]
            return c2
        lax.fori_loop(0, K, row, 0)

        pltpu.sync_copy(hv, mij_hbm.at[pl.ds(base, K)])
        pltpu.sync_copy(hv, acc.at[dstv], add=True)
        return carry

    lax.fori_loop(0, CH, chunk, 0)
    plsc.subcore_barrier()

    pltpu.sync_copy(acc.at[pl.ds(sid * rpt, rpt)],
                    part_hbm.at[cid, pl.ds(sid * rpt, rpt)])
    @pl.when(sid == NS - 1)
    def _():
        pltpu.sync_copy(acc.at[pl.ds(NS * rpt, N - NS * rpt)],
                        part_hbm.at[cid, pl.ds(NS * rpt, N - NS * rpt)])


def _sc_edge(h, wfilt, src, dst):
    mesh = plsc.VectorSubcoreMesh(core_axis_name="c", subcore_axis_name="s")
    zero = jnp.zeros((N, F), dtype=jnp.float32)
    fn = functools.partial(
        pl.kernel,
        mesh=mesh,
        out_type=[
            jax.ShapeDtypeStruct((E, F), jnp.float32),
            jax.ShapeDtypeStruct((NC, N, F), jnp.float32),
        ],
        scratch_types=[
            pltpu.VMEM((K,), jnp.int32),
            pltpu.VMEM((K,), jnp.int32),
            pltpu.VMEM((K, F), jnp.float32),
            pltpu.VMEM((K, F), jnp.float32),
            pltpu.VMEM_SHARED((N, F), jnp.float32),
            pltpu.SemaphoreType.DMA,
        ],
    )(_sc_body)
    return fn(h, wfilt, src, dst, zero)


def _final_body(x_ref, p0_ref, p1_ref, w2t_ref, b2_ref,
                wla_ref, wlb_ref, bl_ref, out_ref):
    m = p0_ref[...] + p1_ref[...]
    t = _ssp(jnp.dot(m, w2t_ref[...],
                     preferred_element_type=jnp.float32) + b2_ref[...])
    out_ref[...] = (jnp.dot(x_ref[...], wla_ref[...],
                            preferred_element_type=jnp.float32)
                    + jnp.dot(t, wlb_ref[...],
                              preferred_element_type=jnp.float32)
                    + bl_ref[...])


def _final(x, p0, p1, W2, b2, Wlin, blin):
    return pl.pallas_call(
        _final_body,
        grid=(N // BN,),
        in_specs=[
            pl.BlockSpec((BN, H), lambda i: (i, 0)),
            pl.BlockSpec((BN, F), lambda i: (i, 0)),
            pl.BlockSpec((BN, F), lambda i: (i, 0)),
            pl.BlockSpec((F, H), lambda i: (0, 0)),
            pl.BlockSpec((1, H), lambda i: (0, 0)),
            pl.BlockSpec((H, H), lambda i: (0, 0)),
            pl.BlockSpec((H, H), lambda i: (0, 0)),
            pl.BlockSpec((1, H), lambda i: (0, 0)),
        ],
        out_specs=pl.BlockSpec((BN, H), lambda i: (i, 0)),
        out_shape=jax.ShapeDtypeStruct((N, H), jnp.float32),
    )(x, p0, p1, W2.T, b2.reshape(1, H), Wlin[:, :H].T, Wlin[:, H:].T,
      blin.reshape(1, H))


def kernel(x, edge_index, edge_length, edge_attr,
           W1, mlp_w1, mlp_b1, mlp_w2, mlp_b2, W2, b2, Wlin, blin):
    src = edge_index[0].astype(jnp.int32)
    dst = edge_index[1].astype(jnp.int32)

    wfilt = _wfilt(edge_attr, edge_length, mlp_w1, mlp_b1, mlp_w2, mlp_b2)
    h = _h(x, W1)
    m_ij, partial = _sc_edge(h, wfilt, src, dst)
    out = _final(x, partial[0], partial[1], W2, b2, Wlin, blin)
    return (out, m_ij)
